# baseline (speedup 1.0000x reference)
import functools

import jax
import jax.numpy as jnp
from jax import lax
from jax.experimental import pallas as pl
from jax.experimental.pallas import tpu as pltpu
from jax.experimental.pallas import tpu_sc as plsc

HIDDEN_DIM = 2048
N_EXPERTS = 64
BLOCK_T = 2048
N_TOKENS = 16384

NC, NS, L = 2, 16, 16
NW = NC * NS
TW = N_TOKENS // NW
GROUPS = TW // L


def _matmul_block(hs_ref, w_ref, logits_ref):
    logits_ref[...] = lax.dot_general(
        hs_ref[...], w_ref[...], (((1,), (1,)), ((), ())),
        preferred_element_type=jnp.float32)


_sc_mesh = plsc.VectorSubcoreMesh(
    core_axis_name="c", subcore_axis_name="s", num_cores=NC, num_subcores=NS)


@functools.partial(
    pl.kernel,
    mesh=_sc_mesh,
    out_type=(
        jax.ShapeDtypeStruct((N_TOKENS * 2,), jnp.float32),
        jax.ShapeDtypeStruct((N_TOKENS * 2,), jnp.int32),
    ),
    scratch_types=[
        pltpu.VMEM((TW * N_EXPERTS,), jnp.float32),
        pltpu.VMEM((TW * 2,), jnp.float32),
        pltpu.VMEM((TW * 2,), jnp.int32),
    ],
    compiler_params=pltpu.CompilerParams(needs_layout_passes=False),
)
def _sc_top2(logits_hbm, scores_hbm, idx_hbm, slab, sc_v, ix_v):
    wid = lax.axis_index("s") * NC + lax.axis_index("c")
    base = wid * TW
    pltpu.sync_copy(logits_hbm.at[pl.ds(base * N_EXPERTS, TW * N_EXPERTS)], slab)
    iota16 = lax.iota(jnp.int32, L)

    def group_body(g, carry):
        tok = g * L + iota16
        flat = tok * N_EXPERTS
        neg_inf = jnp.full((L,), -jnp.inf, jnp.float32)
        zero_i = jnp.zeros((L,), jnp.int32)
        t1v, t2v = neg_inf, neg_inf
        t1i, t2i = zero_i, zero_i
        for e in range(N_EXPERTS):
            ev = jnp.full((L,), e, jnp.int32)
            v = plsc.load_gather(slab, [flat + e])
            gt1 = v > t1v
            gt2 = v > t2v
            t2v = jnp.where(gt1, t1v, jnp.where(gt2, v, t2v))
            t2i = jnp.where(gt1, t1i, jnp.where(gt2, ev, t2i))
            t1v = jnp.where(gt1, v, t1v)
            t1i = jnp.where(gt1, ev, t1i)
        s1 = 1.0 / (1.0 + jnp.exp(t2v - t1v))
        s2 = 1.0 - s1
        two_tok = tok * 2
        plsc.store_scatter(sc_v, [two_tok], s1)
        plsc.store_scatter(sc_v, [two_tok + 1], s2)
        plsc.store_scatter(ix_v, [two_tok], t1i)
        plsc.store_scatter(ix_v, [two_tok + 1], t2i)
        return carry

    lax.fori_loop(0, GROUPS, group_body, 0)
    pltpu.sync_copy(sc_v, scores_hbm.at[pl.ds(base * 2, TW * 2)])
    pltpu.sync_copy(ix_v, idx_hbm.at[pl.ds(base * 2, TW * 2)])


@jax.jit
def kernel(hidden_states, weight):
    hs = hidden_states.reshape(-1, HIDDEN_DIM)
    logits = pl.pallas_call(
        _matmul_block,
        grid=(N_TOKENS // BLOCK_T,),
        in_specs=[
            pl.BlockSpec((BLOCK_T, HIDDEN_DIM), lambda i: (i, 0)),
            pl.BlockSpec((N_EXPERTS, HIDDEN_DIM), lambda i: (0, 0)),
        ],
        out_specs=pl.BlockSpec((BLOCK_T, N_EXPERTS), lambda i: (i, 0)),
        out_shape=jax.ShapeDtypeStruct((N_TOKENS, N_EXPERTS), jnp.float32),
    )(hs, weight)
    scores_flat, indices_flat = _sc_top2(logits.reshape(-1))
    return (logits, scores_flat.reshape(N_TOKENS, 2),
            indices_flat.reshape(N_TOKENS, 2))
